# baseline (speedup 1.0000x reference)
import functools

import jax
import jax.numpy as jnp
from jax import lax
from jax.experimental import pallas as pl
from jax.experimental.pallas import tpu as pltpu
from jax.experimental.pallas import tpu_sc as plsc

D = 64
WINDOW = 8
LANES = 16
BS = 128
G = 2
CHUNK = G * BS
NBUF = 2


def kernel(input, table, b_embed):
    B, N = input.shape
    BN = B * N
    idx_t = input.T.reshape(BN).astype(jnp.int32)

    info = plsc.get_sparse_core_info()
    num_workers = info.num_cores * info.num_subcores
    n_bt = B // BS
    n_btg = n_bt // G
    total_chunks = N * n_btg
    per_w = total_chunks // num_workers
    idx_per_w = per_w * CHUNK
    assert per_w * num_workers == total_chunks and per_w % NBUF == 0

    @functools.partial(
        pl.kernel,
        out_type=jax.ShapeDtypeStruct((N, D // 8, n_bt, 8, BS), jnp.float32),
        mesh=plsc.VectorSubcoreMesh(core_axis_name="c", subcore_axis_name="s"),
        compiler_params=pltpu.CompilerParams(
            use_tc_tiling_on_sc=False, needs_layout_passes=False
        ),
        scratch_types=[
            pltpu.VMEM((idx_per_w,), jnp.int32),
            pltpu.VMEM((NBUF, CHUNK, D), jnp.float32),
            pltpu.VMEM((NBUF, D // 8, G, 8, BS), jnp.float32),
            pltpu.VMEM((D,), jnp.float32),
        ]
        + [pltpu.SemaphoreType.DMA] * (2 * NBUF),
    )
    def body(idx_hbm, table_hbm, bias_hbm, out_hbm, idx_all, rows, obuf, b_v, *sems):
        gsem = sems[:NBUF]
        osem = sems[NBUF:]
        wid = lax.axis_index("s") * info.num_cores + lax.axis_index("c")
        base_c = wid * per_w
        pltpu.sync_copy(bias_hbm, b_v)
        pltpu.sync_copy(idx_hbm.at[pl.ds(base_c * CHUNK, idx_per_w)], idx_all)

        iota16 = lax.iota(jnp.int32, LANES)

        def gather_start(c, p):
            src = table_hbm.at[idx_all.at[pl.ds(c * CHUNK, CHUNK)]]
            pltpu.async_copy(src, rows.at[p], gsem[p])

        def gather_wait(p):
            src = table_hbm.at[idx_all.at[pl.ds(0, CHUNK)]]
            pltpu.make_async_copy(src, rows.at[p], gsem[p]).wait()

        def store_start(n, bt0, q):
            for dt in range(D // 8):
                pltpu.async_copy(
                    obuf.at[q, dt], out_hbm.at[n, dt, pl.ds(bt0, G)], osem[q]
                )

        def store_wait(q):
            for dt in range(D // 8):
                pltpu.make_async_copy(
                    obuf.at[q, dt], out_hbm.at[0, dt, pl.ds(0, G)], osem[q]
                ).wait()

        def add_bias(p):
            def one_row(r, _):
                for k in range(D // LANES):
                    sl = pl.ds(k * LANES, LANES)
                    rows[p, r, sl] = rows[p, r, sl] + b_v[sl]
                return 0

            lax.fori_loop(0, CHUNK, one_row, 0)

        def transpose_chunk(p, q):
            for btl in range(G):
                for dt in range(D // 8):
                    for ds_ in range(8):
                        col = jnp.full((LANES,), dt * 8 + ds_, jnp.int32)
                        for g in range(BS // LANES):
                            row_vec = btl * BS + g * LANES + iota16
                            v = plsc.load_gather(rows.at[p], [row_vec, col])
                            obuf[q, dt, btl, ds_, pl.ds(g * LANES, LANES)] = v

        for p in range(NBUF):
            gather_start(p, p)

        def outer(t, _):
            for p in range(NBUF):
                c = t * NBUF + p
                c_id = base_c + c
                n = c_id // n_btg
                bt0 = (c_id % n_btg) * G
                gather_wait(p)

                @pl.when(n % WINDOW == 0)
                def _():
                    add_bias(p)

                @pl.when(c >= NBUF)
                def _():
                    store_wait(p)

                transpose_chunk(p, p)
                store_start(n, bt0, p)

                @pl.when(c + NBUF < per_w)
                def _():
                    gather_start(c + NBUF, p)

            return 0

        lax.fori_loop(0, per_w // NBUF, outer, 0)
        for q in range(NBUF):
            store_wait(q)

    out5 = body(idx_t, table, b_embed)
    r = jnp.transpose(out5, (0, 1, 3, 2, 4)).reshape(N, D, B)
    return jnp.transpose(r, (2, 0, 1))


# speedup vs baseline: 1.4843x; 1.4843x over previous
import functools

import jax
import jax.numpy as jnp
from jax import lax
from jax.experimental import pallas as pl
from jax.experimental.pallas import tpu as pltpu
from jax.experimental.pallas import tpu_sc as plsc

D = 64
WINDOW = 8
LANES = 16
BS = 128
G = 2
CHUNK = G * BS
NBUF = 2


def kernel(input, table, b_embed):
    B, N = input.shape
    BN = B * N
    idx_t = input.T.reshape(BN).astype(jnp.int32)

    info = plsc.get_sparse_core_info()
    num_workers = info.num_cores * info.num_subcores
    n_bt = B // BS
    n_btg = n_bt // G
    total_chunks = N * n_btg
    per_w = total_chunks // num_workers
    idx_per_w = per_w * CHUNK
    assert per_w * num_workers == total_chunks and per_w % NBUF == 0

    @functools.partial(
        pl.kernel,
        out_type=jax.ShapeDtypeStruct((N, D // 8, n_bt, 8, BS), jnp.float32),
        mesh=plsc.VectorSubcoreMesh(core_axis_name="c", subcore_axis_name="s"),
        compiler_params=pltpu.CompilerParams(
            use_tc_tiling_on_sc=False, needs_layout_passes=False
        ),
        scratch_types=[
            pltpu.VMEM((idx_per_w,), jnp.int32),
            pltpu.VMEM((NBUF, CHUNK, D), jnp.float32),
            pltpu.VMEM((NBUF, D // 8, G, 8, BS), jnp.float32),
            pltpu.VMEM((D,), jnp.float32),
        ]
        + [pltpu.SemaphoreType.DMA] * (2 * NBUF),
    )
    def body(idx_hbm, table_hbm, bias_hbm, out_hbm, idx_all, rows, obuf, b_v, *sems):
        gsem = sems[:NBUF]
        osem = sems[NBUF:]
        wid = lax.axis_index("s") * info.num_cores + lax.axis_index("c")
        base_c = wid * per_w
        pltpu.sync_copy(bias_hbm, b_v)
        pltpu.sync_copy(idx_hbm.at[pl.ds(base_c * CHUNK, idx_per_w)], idx_all)

        iota16 = lax.iota(jnp.int32, LANES)

        def gather_start(c, p):
            src = table_hbm.at[idx_all.at[pl.ds(c * CHUNK, CHUNK)]]
            pltpu.async_copy(src, rows.at[p], gsem[p])

        def gather_wait(p):
            src = table_hbm.at[idx_all.at[pl.ds(0, CHUNK)]]
            pltpu.make_async_copy(src, rows.at[p], gsem[p]).wait()

        def store_start(n, bt0, q):
            for dt in range(D // 8):
                pltpu.async_copy(
                    obuf.at[q, dt], out_hbm.at[n, dt, pl.ds(bt0, G)], osem[q]
                )

        def store_wait(q):
            for dt in range(D // 8):
                pltpu.make_async_copy(
                    obuf.at[q, dt], out_hbm.at[0, dt, pl.ds(0, G)], osem[q]
                ).wait()

        def add_bias(p):
            def one_row(r, _):
                for k in range(D // LANES):
                    sl = pl.ds(k * LANES, LANES)
                    rows[p, r, sl] = rows[p, r, sl] + b_v[sl]
                return 0

            lax.fori_loop(0, CHUNK, one_row, 0)

        def transpose_chunk(p, q):
            @plsc.parallel_loop(0, D, 1, unroll=4)
            def _(c):
                dt = c // 8
                ds_ = c % 8
                col = jnp.full((LANES,), c, jnp.int32)
                for btl in range(G):
                    for g in range(BS // LANES):
                        row_vec = btl * BS + g * LANES + iota16
                        v = plsc.load_gather(rows.at[p], [row_vec, col])
                        obuf[q, dt, btl, ds_, pl.ds(g * LANES, LANES)] = v

        for p in range(NBUF):
            gather_start(p, p)

        def outer(t, _):
            for p in range(NBUF):
                c = t * NBUF + p
                c_id = base_c + c
                n = c_id // n_btg
                bt0 = (c_id % n_btg) * G
                gather_wait(p)

                @pl.when(n % WINDOW == 0)
                def _():
                    add_bias(p)

                @pl.when(c >= NBUF)
                def _():
                    store_wait(p)

                transpose_chunk(p, p)
                store_start(n, bt0, p)

                @pl.when(c + NBUF < per_w)
                def _():
                    gather_start(c + NBUF, p)

            return 0

        lax.fori_loop(0, per_w // NBUF, outer, 0)
        for q in range(NBUF):
            store_wait(q)

    out5 = body(idx_t, table, b_embed)
    r = jnp.transpose(out5, (0, 1, 3, 2, 4)).reshape(N, D, B)
    return jnp.transpose(r, (2, 0, 1))
